# Initial kernel scaffold; baseline (speedup 1.0000x reference)
import jax
import jax.numpy as jnp
from jax.experimental import pallas as pl


def kernel(x, table):
    raise NotImplementedError("write your pallas kernel here")


# speedup vs baseline: 1.5607x; 1.5607x over previous
import functools

import jax
import jax.numpy as jnp
from jax import lax
from jax.experimental import pallas as pl
from jax.experimental.pallas import tpu as pltpu
from jax.experimental.pallas import tpu_sc as plsc

NUM_EMB = 1000000
DIM = 32
BATCH = 16384
FIELDS = 26
B = BATCH * FIELDS
NW = 32
BPW = B // NW
CH = 1024
NCH = BPW // CH

_mesh = plsc.VectorSubcoreMesh(core_axis_name="c", subcore_axis_name="s")


@functools.partial(
    pl.kernel,
    mesh=_mesh,
    out_type=jax.ShapeDtypeStruct((B, DIM), jnp.float32),
    scratch_types=[
        pltpu.VMEM((BPW,), jnp.int32),
        pltpu.VMEM((CH, DIM), jnp.float32),
        pltpu.SemaphoreType.DMA,
    ],
    compiler_params=pltpu.CompilerParams(use_tc_tiling_on_sc=False),
)
def _gather_kernel(idx_hbm, table_hbm, out_hbm, idx_v, rows_v, sem):
    wid = lax.axis_index("s") * 2 + lax.axis_index("c")
    base = wid * BPW
    pltpu.sync_copy(idx_hbm.at[pl.ds(base, BPW)], idx_v)

    def body(c, carry):
        idx_chunk = idx_v.at[pl.ds(c * CH, CH)]
        pltpu.async_copy(table_hbm.at[idx_chunk], rows_v, sem).wait()
        pltpu.sync_copy(rows_v, out_hbm.at[pl.ds(base + c * CH, CH)])
        return carry

    lax.fori_loop(0, NCH, body, 0)


def kernel(x, table):
    flat_idx = x.reshape(B)
    out = _gather_kernel(flat_idx, table)
    return out.reshape(BATCH, FIELDS, DIM)


# speedup vs baseline: 1.5721x; 1.0073x over previous
import functools

import jax
import jax.numpy as jnp
from jax import lax
from jax.experimental import pallas as pl
from jax.experimental.pallas import tpu as pltpu
from jax.experimental.pallas import tpu_sc as plsc

NUM_EMB = 1000000
DIM = 32
BATCH = 16384
FIELDS = 26
B = BATCH * FIELDS
NW = 32
BPW = B // NW
NB = 4
CH = 832
NCH = BPW // CH

_mesh = plsc.VectorSubcoreMesh(core_axis_name="c", subcore_axis_name="s")


@functools.partial(
    pl.kernel,
    mesh=_mesh,
    out_type=jax.ShapeDtypeStruct((B, DIM), jnp.float32),
    scratch_types=[
        pltpu.VMEM((BPW,), jnp.int32),
        pltpu.VMEM((NB, CH, DIM), jnp.float32),
        pltpu.SemaphoreType.DMA((NB,)),
        pltpu.SemaphoreType.DMA((NB,)),
    ],
    compiler_params=pltpu.CompilerParams(use_tc_tiling_on_sc=False),
)
def _gather_kernel(idx_hbm, table_hbm, out_hbm, idx_v, bufs, gsem, ssem):
    wid = lax.axis_index("s") * 2 + lax.axis_index("c")
    base = wid * BPW
    pltpu.sync_copy(idx_hbm.at[pl.ds(base, BPW)], idx_v)

    gat = [None] * NCH
    scat = [None] * NCH

    def issue_gather(c):
        b = c % NB
        gat[c] = pltpu.async_copy(
            table_hbm.at[idx_v.at[pl.ds(c * CH, CH)]], bufs.at[b], gsem.at[b])

    issue_gather(0)
    for c in range(NCH):
        if c + 1 < NCH:
            if c + 1 >= NB:
                scat[c + 1 - NB].wait()
            issue_gather(c + 1)
        gat[c].wait()
        b = c % NB
        scat[c] = pltpu.async_copy(
            bufs.at[b], out_hbm.at[pl.ds(base + c * CH, CH)], ssem.at[b])
    for c in range(max(0, NCH - NB), NCH):
        scat[c].wait()


def kernel(x, table):
    flat_idx = x.reshape(B)
    out = _gather_kernel(flat_idx, table)
    return out.reshape(BATCH, FIELDS, DIM)


# speedup vs baseline: 1.6650x; 1.0591x over previous
import functools

import jax
import jax.numpy as jnp
from jax import lax
from jax.experimental import pallas as pl
from jax.experimental.pallas import tpu as pltpu
from jax.experimental.pallas import tpu_sc as plsc

NUM_EMB = 1000000
DIM = 32
BATCH = 16384
FIELDS = 26
B = BATCH * FIELDS
NW = 32
BPW = B // NW
NB = 4
CH = 832
NCH = BPW // CH

_mesh = plsc.VectorSubcoreMesh(core_axis_name="c", subcore_axis_name="s")


@functools.partial(
    pl.kernel,
    mesh=_mesh,
    out_type=jax.ShapeDtypeStruct((B, DIM), jnp.float32),
    scratch_types=[
        pltpu.VMEM((BPW,), jnp.int32),
        pltpu.VMEM((NB, CH, DIM), jnp.float32),
        pltpu.SemaphoreType.DMA((NB,)),
        pltpu.SemaphoreType.DMA((NB,)),
    ],
    compiler_params=pltpu.CompilerParams(use_tc_tiling_on_sc=False),
)
def _gather_kernel(idx_hbm, table_hbm, out_hbm, idx_v, bufs, gsem, ssem):
    wid = lax.axis_index("s") * 2 + lax.axis_index("c")
    base = wid * BPW
    pltpu.sync_copy(idx_hbm.at[pl.ds(base, BPW)], idx_v)

    gat = [None] * NCH
    scat = [None] * NCH

    def issue_gather(c):
        b = c % NB
        gat[c] = pltpu.async_copy(
            table_hbm.at[idx_v.at[pl.ds(c * CH, CH)]], bufs.at[b], gsem.at[b])

    issue_gather(0)
    for c in range(NCH):
        if c + 1 < NCH:
            if c + 1 >= NB:
                scat[c + 1 - NB].wait()
            issue_gather(c + 1)
        gat[c].wait()
        b = c % NB
        scat[c] = pltpu.async_copy(
            bufs.at[b], out_hbm.at[pl.ds(base + c * CH, CH)], ssem.at[b])
    for c in range(max(0, NCH - NB), NCH):
        scat[c].wait()


def kernel(x, table):
    flat_idx = x.T.reshape(B)
    out = _gather_kernel(flat_idx, table)
    return out.reshape(FIELDS, BATCH, DIM).transpose(1, 0, 2)


# speedup vs baseline: 1.6662x; 1.0007x over previous
import functools

import jax
import jax.numpy as jnp
from jax import lax
from jax.experimental import pallas as pl
from jax.experimental.pallas import tpu as pltpu
from jax.experimental.pallas import tpu_sc as plsc

NUM_EMB = 1000000
DIM = 32
BATCH = 16384
FIELDS = 26
B = BATCH * FIELDS
NW = 32
BPW = B // NW
NB = 4
CH = 832
NCH = BPW // CH

_mesh = plsc.VectorSubcoreMesh(core_axis_name="c", subcore_axis_name="s")


@functools.partial(
    pl.kernel,
    mesh=_mesh,
    out_type=jax.ShapeDtypeStruct((B, DIM), jnp.float32),
    scratch_types=[
        pltpu.VMEM((BPW,), jnp.int32),
        pltpu.VMEM((NB, CH, DIM), jnp.float32),
        pltpu.SemaphoreType.DMA((NB,)),
        pltpu.SemaphoreType.DMA((NB,)),
    ],
    compiler_params=pltpu.CompilerParams(use_tc_tiling_on_sc=False),
)
def _gather_kernel(idx_hbm, table_hbm, out_hbm, idx_v, bufs, gsem, ssem):
    wid = lax.axis_index("s") * 2 + lax.axis_index("c")
    base = wid * BPW
    pltpu.sync_copy(idx_hbm.at[pl.ds(base, BPW)], idx_v)

    gat = [None] * NCH
    scat = [None] * NCH

    def issue_gather(c):
        b = c % NB
        gat[c] = pltpu.async_copy(
            table_hbm.at[idx_v.at[pl.ds(c * CH, CH)]], bufs.at[b], gsem.at[b])

    issue_gather(0)
    for c in range(NCH):
        if c + 1 < NCH:
            if c + 1 >= NB:
                scat[c + 1 - NB].wait()
            issue_gather(c + 1)
        gat[c].wait()
        b = c % NB
        scat[c] = pltpu.async_copy(
            bufs.at[b], out_hbm.at[pl.ds(base + c * CH, CH)], ssem.at[b])
    for c in range(max(0, NCH - NB), NCH):
        scat[c].wait()


@functools.partial(
    pl.kernel,
    mesh=_mesh,
    out_type=jax.ShapeDtypeStruct((B,), jnp.int32),
    scratch_types=[pltpu.VMEM((BATCH,), jnp.int32)],
    compiler_params=pltpu.CompilerParams(use_tc_tiling_on_sc=True),
)
def _detile_idx(xt_hbm, flat_hbm, buf):
    wid = lax.axis_index("s") * 2 + lax.axis_index("c")

    @pl.when(wid < FIELDS)
    def _():
        pltpu.sync_copy(xt_hbm.at[wid], buf)
        pltpu.sync_copy(buf, flat_hbm.at[pl.ds(wid * BATCH, BATCH)])


def kernel(x, table):
    flat_idx = _detile_idx(x.T)
    out = _gather_kernel(flat_idx, table)
    return out.reshape(FIELDS, BATCH, DIM).transpose(1, 0, 2)


# speedup vs baseline: 1.6688x; 1.0016x over previous
import functools

import jax
import jax.numpy as jnp
from jax import lax
from jax.experimental import pallas as pl
from jax.experimental.pallas import tpu as pltpu
from jax.experimental.pallas import tpu_sc as plsc

NUM_EMB = 1000000
DIM = 32
BATCH = 16384
FIELDS = 26
B = BATCH * FIELDS
NW = 32
BPW = B // NW
NB = 4
CH = 832
NCH = BPW // CH

_mesh = plsc.VectorSubcoreMesh(core_axis_name="c", subcore_axis_name="s")


@functools.partial(
    pl.kernel,
    mesh=_mesh,
    out_type=jax.ShapeDtypeStruct((B, DIM), jnp.float32),
    scratch_types=[
        pltpu.VMEM((BPW,), jnp.int32),
        pltpu.VMEM((NB, CH, DIM), jnp.float32),
        pltpu.SemaphoreType.DMA((NB,)),
        pltpu.SemaphoreType.DMA((NB,)),
    ],
    compiler_params=pltpu.CompilerParams(use_tc_tiling_on_sc=False),
)
def _gather_kernel(idx_hbm, table_hbm, out_hbm, idx_v, bufs, gsem, ssem):
    wid = lax.axis_index("s") * 2 + lax.axis_index("c")
    base = wid * BPW
    pltpu.sync_copy(idx_hbm.at[pl.ds(base, BPW)], idx_v)

    gat = [None] * NCH
    scat = [None] * NCH

    def issue_gather(c):
        b = c % NB
        gat[c] = pltpu.async_copy(
            table_hbm.at[idx_v.at[pl.ds(c * CH, CH)]], bufs.at[b], gsem.at[b])

    issue_gather(0)
    for c in range(NCH):
        if c + 1 < NCH:
            if c + 1 >= NB:
                scat[c + 1 - NB].wait()
            issue_gather(c + 1)
        gat[c].wait()
        b = c % NB
        scat[c] = pltpu.async_copy(
            bufs.at[b], out_hbm.at[pl.ds(base + c * CH, CH)], ssem.at[b])
    for c in range(max(0, NCH - NB), NCH):
        scat[c].wait()


@functools.partial(
    pl.kernel,
    mesh=_mesh,
    out_type=jax.ShapeDtypeStruct((B,), jnp.int32),
    scratch_types=[pltpu.VMEM((BATCH,), jnp.int32)],
    compiler_params=pltpu.CompilerParams(use_tc_tiling_on_sc=True),
)
def _detile_idx(xt_hbm, flat_hbm, buf):
    wid = lax.axis_index("s") * 2 + lax.axis_index("c")

    @pl.when(wid < FIELDS)
    def _():
        pltpu.sync_copy(xt_hbm.at[wid], buf)
        pltpu.sync_copy(buf, flat_hbm.at[pl.ds(wid * BATCH, BATCH)])


def kernel(x, table):
    flat_idx = _detile_idx(x.T)
    table_wide = jax.lax.optimization_barrier(table.reshape(NUM_EMB // 4, 4 * DIM))
    out = _gather_kernel(flat_idx, table_wide.reshape(NUM_EMB, DIM))
    return out.reshape(FIELDS, BATCH, DIM).transpose(1, 0, 2)
